# baseline (speedup 1.0000x reference)
import functools

import jax
import jax.numpy as jnp
import numpy as np
from jax import lax
from jax.experimental import pallas as pl
from jax.experimental.pallas import tpu as pltpu
from jax.experimental.pallas import tpu_sc as plsc

GRID = 32
BASE = 128
SENT = 4 * BASE ** 3
FILL = 8 * BASE ** 3
N = 10000
N_PAD = 10240
NW = 32

OFF27 = np.array([[dx, dy, dz] for dx in (-1, 0, 1) for dy in (-1, 0, 1) for dz in (-1, 0, 1)],
                 dtype=np.int32)


def _encode(b, xyz):
    b = b.astype(jnp.int64)
    x = xyz[:, 0].astype(jnp.int64)
    y = xyz[:, 1].astype(jnp.int64)
    z = xyz[:, 2].astype(jnp.int64)
    return ((b * BASE + x) * BASE + y) * BASE + z


def _build_level_maps(coords, b):
    keys = _encode(b, coords)
    order = jnp.argsort(keys)
    keys = keys[order]
    coords = coords[order]
    b = b[order]
    n = keys.shape[0]
    nb = []
    for k in range(27):
        q = _encode(b, coords + jnp.asarray(OFF27[k])[None, :])
        pos = jnp.searchsorted(keys, q)
        pos_c = jnp.clip(pos, 0, n - 1)
        valid = (pos < n) & (keys[pos_c] == q)
        nb.append(jnp.where(valid, pos_c, n))
    return keys, coords, b, jnp.stack(nb), order


def _build_structure(coords, batch_idx):
    n = coords.shape[0]
    rows = jnp.arange(n, dtype=jnp.int32)
    sent_c = jnp.stack([rows // (BASE * BASE), (rows // BASE) % BASE, rows % BASE], axis=1).astype(jnp.int32)
    sent_b = jnp.full((n,), 4, jnp.int32)
    keys, cc, cb, nb0, order0 = _build_level_maps(coords, batch_idx)
    mask = keys < SENT
    nbs = [nb0]
    masks = [mask]
    downs = []
    for _ in range(4):
        pxyz = cc // 2
        pkey = jnp.where(mask, _encode(cb, pxyz), SENT + rows)
        ukeys, first = jnp.unique(pkey, return_index=True, size=n, fill_value=FILL)
        pidx = jnp.searchsorted(ukeys, pkey)
        off = ((cc[:, 0] % 2) * 4 + (cc[:, 1] % 2) * 2 + (cc[:, 2] % 2)).astype(jnp.int32)
        downs.append((pidx, off, n))
        pmask = ukeys < SENT
        pc = jnp.where(pmask[:, None], pxyz[first], sent_c)
        pb = jnp.where(pmask, cb[first], sent_b)
        keys, cc, cb, nbm, _ = _build_level_maps(pc, pb)
        mask = keys < SENT
        nbs.append(nbm)
        masks.append(mask)
    return order0, nbs, downs, masks


def _pick_chunk(bpw, C, budget=460 * 1024):
    chunk = 8
    for c in range(8, min(bpw, 128) + 1, 8):
        if bpw % c == 0 and c * C * 4 + bpw * 4 <= budget:
            chunk = c
    return chunk


@functools.lru_cache(maxsize=None)
def _make_sc_gather(B, C):
    bpw = B // NW
    chunk = _pick_chunk(bpw, C)
    iters = bpw // chunk
    mesh = plsc.VectorSubcoreMesh(core_axis_name="c", subcore_axis_name="s")

    @functools.partial(
        pl.kernel,
        mesh=mesh,
        out_type=jax.ShapeDtypeStruct((B, C), jnp.float32),
        scratch_types=[
            pltpu.VMEM((bpw,), jnp.int32),
            pltpu.VMEM((chunk, C), jnp.float32),
            pltpu.SemaphoreType.DMA,
        ],
    )
    def gather_k(table_hbm, idx_hbm, out_hbm, idx_v, rows_v, sem):
        wid = lax.axis_index("s") * 2 + lax.axis_index("c")
        base = wid * bpw
        pltpu.sync_copy(idx_hbm.at[pl.ds(base, bpw)], idx_v)

        def body(j, carry):
            off = j * chunk
            pltpu.async_copy(table_hbm.at[idx_v.at[pl.ds(off, chunk)]], rows_v, sem).wait()
            pltpu.sync_copy(rows_v, out_hbm.at[pl.ds(base + off, chunk)])
            return carry

        lax.fori_loop(0, iters, body, 0)

    return gather_k


def _sc_gather(table, idx_flat):
    B = idx_flat.shape[0]
    C = table.shape[1]
    return _make_sc_gather(B, C)(table, idx_flat)


def _mm_body(g_ref, w_ref, b_ref, m_ref, o_ref, s_ref):
    i = pl.program_id(0)
    k = pl.program_id(1)
    acc = jnp.dot(g_ref[0], w_ref[0], preferred_element_type=jnp.float32)

    @pl.when(k == 0)
    def _():
        o_ref[...] = acc + b_ref[0:1, :]

    @pl.when(k != 0)
    def _():
        o_ref[...] = o_ref[...] + acc

    @pl.when((i == 0) & (k == 0))
    def _():
        s_ref[...] = jnp.zeros_like(s_ref)

    @pl.when(k == pl.num_programs(1) - 1)
    def _():
        x = o_ref[...]
        xm = x * m_ref[:, 0:1]
        s_ref[0:1, :] = s_ref[0:1, :] + jnp.sum(xm, axis=0, keepdims=True)
        s_ref[1:2, :] = s_ref[1:2, :] + jnp.sum(xm * x, axis=0, keepdims=True)


def _tc_convmm(G3, W, bias, maskf, bn_rows=1024):
    K, NP, cin = G3.shape
    cout = W.shape[2]
    out, stats = pl.pallas_call(
        _mm_body,
        grid=(NP // bn_rows, K),
        in_specs=[
            pl.BlockSpec((1, bn_rows, cin), lambda i, k: (k, i, 0)),
            pl.BlockSpec((1, cin, cout), lambda i, k: (k, 0, 0)),
            pl.BlockSpec((8, cout), lambda i, k: (0, 0)),
            pl.BlockSpec((bn_rows, 128), lambda i, k: (i, 0)),
        ],
        out_specs=[
            pl.BlockSpec((bn_rows, cout), lambda i, k: (i, 0)),
            pl.BlockSpec((8, cout), lambda i, k: (0, 0)),
        ],
        out_shape=[
            jax.ShapeDtypeStruct((NP, cout), jnp.float32),
            jax.ShapeDtypeStruct((8, cout), jnp.float32),
        ],
    )(G3, W, bias, maskf)
    return out, stats


def _apply_body(x_ref, s_ref, t_ref, m_ref, o_ref):
    x = x_ref[...]
    y = jnp.maximum(x * s_ref[0:1, :] + t_ref[0:1, :], 0.0)
    o_ref[...] = y * m_ref[:, 0:1]


def _tc_apply(x, scale, shift, maskf, bn_rows=1024):
    NP, cout = x.shape
    return pl.pallas_call(
        _apply_body,
        grid=(NP // bn_rows,),
        in_specs=[
            pl.BlockSpec((bn_rows, cout), lambda i: (i, 0)),
            pl.BlockSpec((8, cout), lambda i: (0, 0)),
            pl.BlockSpec((8, cout), lambda i: (0, 0)),
            pl.BlockSpec((bn_rows, 128), lambda i: (i, 0)),
        ],
        out_specs=pl.BlockSpec((bn_rows, cout), lambda i: (i, 0)),
        out_shape=jax.ShapeDtypeStruct((NP, cout), jnp.float32),
    )(x, scale, shift, maskf)


_ZB = {}


def _zero_bias(cout):
    if cout not in _ZB:
        _ZB[cout] = jnp.zeros((8, cout), jnp.float32)
    return _ZB[cout]


def _gmm(table, idx, W, bias=None, maskf=None):
    K, NP = idx.shape
    cin = table.shape[1]
    cout = W.shape[2]
    G = _sc_gather(table, idx.reshape(-1))
    G3 = G.reshape(K, NP, cin)
    if bias is None:
        bias = _zero_bias(cout)
    return _tc_convmm(G3, W, bias, maskf)


def _bn_finalize(stats, g, b, cnt):
    mu = stats[0] / cnt
    var = stats[1] / cnt - mu * mu
    scale = g * lax.rsqrt(var + 1e-5)
    shift = b - mu * scale
    return (jnp.broadcast_to(scale[None, :], (8, scale.shape[0])),
            jnp.broadcast_to(shift[None, :], (8, shift.shape[0])))


def _conv_bn(table, idx, W, g, b, maskf, cnt):
    out, stats = _gmm(table, idx, W, maskf=maskf)
    scale, shift = _bn_finalize(stats, g, b, cnt)
    return _tc_apply(out, scale, shift, maskf)


def _pw(W, cin_p, cout_p):
    K, ci, co = W.shape
    if ci == cin_p and co == cout_p:
        return W
    return jnp.zeros((K, cin_p, cout_p), jnp.float32).at[:, :ci, :co].set(W)


def _pv(v, cout_p):
    if v.shape[0] == cout_p:
        return v
    return jnp.zeros((cout_p,), jnp.float32).at[:v.shape[0]].set(v)


def _conv_block(table, idx, p, maskf, cnt, w1=None, idx2=None):
    W1 = w1 if w1 is not None else p['W1']
    cout_p = max(128, p['W2'].shape[2])
    f = _conv_bn(table, idx, W1, _pv(p['g1'], cout_p), _pv(p['b1'], cout_p), maskf, cnt)
    W2 = _pw(p['W2'], cout_p, cout_p)
    f = _conv_bn(f, idx if idx2 is None else idx2, W2,
                 _pv(p['g2'], cout_p), _pv(p['b2'], cout_p), maskf, cnt)
    return f


def _pad_idx27(nbm):
    idx = jnp.full((27, N_PAD), N, jnp.int32)
    return idx.at[:, :N].set(nbm.astype(jnp.int32))


def _down_idx(pidx, off):
    ar = jnp.arange(N, dtype=jnp.int32)
    idx = jnp.full((8, N_PAD), N, jnp.int32)
    return idx.at[off, pidx].set(ar)


def _up_idx(pidx, off):
    o = jnp.arange(8, dtype=jnp.int32)[:, None]
    sel = jnp.where(off[None, :] == o, pidx[None, :].astype(jnp.int32), N)
    idx = jnp.full((8, N_PAD), N, jnp.int32)
    return idx.at[:, :N].set(sel)


def _maskf_of(mask):
    m = jnp.zeros((N_PAD,), jnp.float32).at[:N].set(mask.astype(jnp.float32))
    return jnp.broadcast_to(m[:, None], (N_PAD, 128))


def kernel(feats, params, coords, batch_idx):
    order0, nbs, downs, masks = _build_structure(coords, batch_idx)

    cnts = [jnp.sum(m.astype(jnp.float32)) for m in masks]
    maskfs = [_maskf_of(m) for m in masks]
    idx27 = [_pad_idx27(nb) for nb in nbs]
    idx_dn = [_down_idx(pidx, off) for (pidx, off, _) in downs]
    idx_up = [_up_idx(pidx, off) for (pidx, off, _) in downs]

    p = params

    def dec_block(u, e, idx, pdec, maskf, cnt):
        c_u = u.shape[1]
        table = jnp.concatenate([u, e], axis=0)
        idx54 = jnp.concatenate([idx, idx + N_PAD], axis=0)
        W1 = pdec['W1']
        ci = W1.shape[1] // 2
        cout_p = max(128, W1.shape[2])
        W54 = jnp.concatenate(
            [_pw(W1[:, :ci, :], c_u, cout_p), _pw(W1[:, ci:, :], c_u, cout_p)], axis=0)
        return _conv_block(table, idx54, pdec, maskf, cnt, w1=W54, idx2=idx)

    f0 = jnp.zeros((N_PAD, 128), jnp.float32).at[:N, :4].set(feats[order0])

    e1 = _conv_block(f0, idx27[0], p['enc1'], maskfs[0], cnts[0], w1=_pw(p['enc1']['W1'], 128, 128))
    x2, _ = _gmm(e1, idx_dn[0], _pw(p['down1'], 128, 128), maskf=maskfs[1])
    e2 = _conv_block(x2, idx27[1], p['enc2'], maskfs[1], cnts[1], w1=_pw(p['enc2']['W1'], 128, 128))
    x3, _ = _gmm(e2, idx_dn[1], p['down2'], maskf=maskfs[2])
    e3 = _conv_block(x3, idx27[2], p['enc3'], maskfs[2], cnts[2])
    x4, _ = _gmm(e3, idx_dn[2], p['down3'], maskf=maskfs[3])
    e4 = _conv_block(x4, idx27[3], p['enc4'], maskfs[3], cnts[3])
    x5, _ = _gmm(e4, idx_dn[3], p['down4'], maskf=maskfs[4])
    bck = _conv_block(x5, idx27[4], p['bottleneck'], maskfs[4], cnts[4])

    u4, _ = _gmm(bck, idx_up[3], p['up4'], maskf=maskfs[3])
    d4 = dec_block(u4, e4, idx27[3], p['dec4'], maskfs[3], cnts[3])
    u3, _ = _gmm(d4, idx_up[2], p['up3'], maskf=maskfs[2])
    d3 = dec_block(u3, e3, idx27[2], p['dec3'], maskfs[2], cnts[2])
    u2, _ = _gmm(d3, idx_up[1], p['up2'], maskf=maskfs[1])
    d2 = dec_block(u2, e2, idx27[1], p['dec2'], maskfs[1], cnts[1])
    u1, _ = _gmm(d2, idx_up[0], _pw(p['up1'], 128, 128), maskf=maskfs[0])
    d1 = dec_block(u1, e1, idx27[0], p['dec1'], maskfs[0], cnts[0])

    Wf = jnp.zeros((1, 128, 128), jnp.float32).at[0, :64, :2].set(p['final']['W'])
    bf = jnp.broadcast_to(
        jnp.zeros((128,), jnp.float32).at[:2].set(p['final']['b'])[None, :], (8, 128))
    out, _ = _tc_convmm(d1.reshape(1, N_PAD, 128), Wf, bf, maskfs[0])
    return out[:N, :2]
